# baseline (speedup 1.0000x reference)
import jax
import jax.numpy as jnp
from jax.experimental import pallas as pl
from jax.experimental.pallas import tpu as pltpu

_BLK = 512


def _body(idx_ref, elem_ref, arr_ref, out_ref):
    i = pl.program_id(0)
    local = idx_ref[0] - i * _BLK
    rows = jax.lax.broadcasted_iota(jnp.int32, (_BLK, 1), 0)
    out_ref[...] = jnp.where(rows == local, elem_ref[...], arr_ref[...])


def kernel(arr, index, element):
    M, N = arr.shape
    idx = jnp.asarray(index, jnp.int32).reshape((1,))
    elem2d = element.reshape((1, N))
    return pl.pallas_call(
        _body,
        grid_spec=pltpu.PrefetchScalarGridSpec(
            num_scalar_prefetch=1,
            grid=(M // _BLK,),
            in_specs=[
                pl.BlockSpec((1, N), lambda i, idx_ref: (0, 0)),
                pl.BlockSpec((_BLK, N), lambda i, idx_ref: (i, 0)),
            ],
            out_specs=pl.BlockSpec((_BLK, N), lambda i, idx_ref: (i, 0)),
        ),
        out_shape=jax.ShapeDtypeStruct((M, N), arr.dtype),
    )(idx, elem2d, arr)
